# baseline (speedup 1.0000x reference)
import functools

import jax
import jax.numpy as jnp
from jax import lax
from jax.experimental import pallas as pl
from jax.experimental.pallas import tpu as pltpu
from jax.experimental.pallas import tpu_sc as plsc

_ALPHA = 0.25

_ROWS = 1024
_COLS = 16384
_BLOCK_ROWS = 64

_SC_ROWS = 256

_NC, _NS, _LANES = 2, 16, 16
_NW = _NC * _NS
_SC_N = _SC_ROWS * _COLS
_PER_WORKER = _SC_N // _NW
_CHUNK = 8192
_N_CHUNKS = _PER_WORKER // _CHUNK
_VEC_PER_CHUNK = _CHUNK // _LANES


def _tc_body(pred_ref, target_ref, out_ref, acc_ref):
    i = pl.program_id(0)

    @pl.when(i == 0)
    def _init():
        acc_ref[0] = 0.0
        acc_ref[1] = 0.0

    x = pred_ref[...]
    label = target_ref[...]
    s = x * (1.0 - 2.0 * label)
    w = (1.0 - _ALPHA) - (1.0 - 2.0 * _ALPHA) * label
    e = jnp.exp(-jnp.abs(s))
    denom = 1.0 + e
    inv = 1.0 / denom
    q = jnp.where(s >= 0.0, inv, 1.0 - inv)
    sp = jnp.maximum(s, 0.0) + jnp.log(denom)
    acc_ref[0] += jnp.sum(w * (q * q) * sp)
    acc_ref[1] += jnp.sum(label)

    @pl.when(i == pl.num_programs(0) - 1)
    def _finish():
        out_ref[0] = acc_ref[0]
        out_ref[1] = acc_ref[1]


def _tc_partials(pred, target):
    n_rows = _ROWS - _SC_ROWS
    grid = n_rows // _BLOCK_ROWS
    row0 = _SC_ROWS // _BLOCK_ROWS
    return pl.pallas_call(
        _tc_body,
        grid=(grid,),
        in_specs=[
            pl.BlockSpec((_BLOCK_ROWS, _COLS), lambda i: (i + row0, 0)),
            pl.BlockSpec((_BLOCK_ROWS, _COLS), lambda i: (i + row0, 0)),
        ],
        out_specs=pl.BlockSpec(memory_space=pltpu.SMEM),
        out_shape=jax.ShapeDtypeStruct((2,), jnp.float32),
        scratch_shapes=[pltpu.SMEM((2,), jnp.float32)],
    )(pred, target)


def _sc_body(pred_hbm, target_hbm, loss_hbm, pos_hbm, px, pt, accv, sem):
    wid = lax.axis_index("s") * _NC + lax.axis_index("c")
    base0 = wid * _PER_WORKER

    def chunk_body(j, carry):
        acc_loss, acc_pos = carry
        base = base0 + j * _CHUNK
        pltpu.sync_copy(pred_hbm.at[pl.ds(base, _CHUNK)], px)
        pltpu.sync_copy(target_hbm.at[pl.ds(base, _CHUNK)], pt)

        def vec_body(v, c2):
            al, ap = c2
            x = px[pl.ds(v * _LANES, _LANES)]
            lab = pt[pl.ds(v * _LANES, _LANES)]
            s = x * (1.0 - 2.0 * lab)
            e = jnp.exp(-jnp.abs(x))
            inv = 1.0 / (1.0 + e)
            q = jnp.where(s >= 0.0, inv, 1.0 - inv)
            t = e / (2.0 + e)
            t2 = t * t
            poly = 1.0 + t2 * (1.0 / 3.0 + t2 * (1.0 / 5.0 + t2 * (
                1.0 / 7.0 + t2 * (1.0 / 9.0))))
            sp = jnp.maximum(s, 0.0) + 2.0 * t * poly
            w = 0.75 - 0.5 * lab
            return (al + w * (q * q) * sp, ap + lab)

        return lax.fori_loop(0, _VEC_PER_CHUNK, vec_body, (acc_loss, acc_pos))

    zero = jnp.zeros((_LANES,), jnp.float32)
    acc_loss, acc_pos = lax.fori_loop(0, _N_CHUNKS, chunk_body, (zero, zero))
    accv[pl.ds(0, _LANES)] = acc_loss
    accv[pl.ds(_LANES, _LANES)] = acc_pos
    pltpu.sync_copy(accv.at[pl.ds(0, _LANES)],
                    loss_hbm.at[pl.ds(wid * _LANES, _LANES)])
    pltpu.sync_copy(accv.at[pl.ds(_LANES, _LANES)],
                    pos_hbm.at[pl.ds(wid * _LANES, _LANES)])


def _sc_partials(pred, target):
    mesh = plsc.VectorSubcoreMesh(
        core_axis_name="c", subcore_axis_name="s",
        num_cores=_NC, num_subcores=_NS)
    k = pl.kernel(
        _sc_body,
        out_type=(
            jax.ShapeDtypeStruct((_NW * _LANES,), jnp.float32),
            jax.ShapeDtypeStruct((_NW * _LANES,), jnp.float32),
        ),
        mesh=mesh,
        scratch_types=[
            pltpu.VMEM((_CHUNK,), jnp.float32),
            pltpu.VMEM((_CHUNK,), jnp.float32),
            pltpu.VMEM((2 * _LANES,), jnp.float32),
            pltpu.SemaphoreType.DMA,
        ],
    )
    return k(pred.reshape(-1), target.reshape(-1))


def kernel(pred, target):
    sc_loss, sc_pos = _sc_partials(pred, target)
    tc_out = _tc_partials(pred, target)
    loss_sum = tc_out[0] + jnp.sum(sc_loss)
    pos_num = jnp.maximum(tc_out[1] + jnp.sum(sc_pos), 1.0)
    return loss_sum / pos_num


# speedup vs baseline: 2.1608x; 2.1608x over previous
import functools

import jax
import jax.numpy as jnp
from jax import lax
from jax.experimental import pallas as pl
from jax.experimental.pallas import tpu as pltpu
from jax.experimental.pallas import tpu_sc as plsc

_ALPHA = 0.25

_ROWS = 1024
_COLS = 16384
_BLOCK_ROWS = 64

_SC_ROWS = 256

_NC, _NS, _LANES = 2, 16, 16
_NW = _NC * _NS
_ROWS_PER_WORKER = _SC_ROWS // _NW
_CHUNK_COLS = 4096
_N_CHUNKS = _COLS // _CHUNK_COLS
_VEC_PER_CHUNK = _CHUNK_COLS // _LANES


def _tc_body(pred_ref, target_ref, out_ref, acc_ref):
    i = pl.program_id(0)

    @pl.when(i == 0)
    def _init():
        acc_ref[0] = 0.0
        acc_ref[1] = 0.0

    x = pred_ref[...]
    label = target_ref[...]
    s = x * (1.0 - 2.0 * label)
    w = (1.0 - _ALPHA) - (1.0 - 2.0 * _ALPHA) * label
    e = jnp.exp(-jnp.abs(s))
    denom = 1.0 + e
    inv = 1.0 / denom
    q = jnp.where(s >= 0.0, inv, 1.0 - inv)
    sp = jnp.maximum(s, 0.0) + jnp.log(denom)
    acc_ref[0] += jnp.sum(w * (q * q) * sp)
    acc_ref[1] += jnp.sum(label)

    @pl.when(i == pl.num_programs(0) - 1)
    def _finish():
        out_ref[0] = acc_ref[0]
        out_ref[1] = acc_ref[1]


def _tc_partials(pred, target):
    n_rows = _ROWS - _SC_ROWS
    grid = n_rows // _BLOCK_ROWS
    row0 = _SC_ROWS // _BLOCK_ROWS
    return pl.pallas_call(
        _tc_body,
        grid=(grid,),
        in_specs=[
            pl.BlockSpec((_BLOCK_ROWS, _COLS), lambda i: (i + row0, 0)),
            pl.BlockSpec((_BLOCK_ROWS, _COLS), lambda i: (i + row0, 0)),
        ],
        out_specs=pl.BlockSpec(memory_space=pltpu.SMEM),
        out_shape=jax.ShapeDtypeStruct((2,), jnp.float32),
        scratch_shapes=[pltpu.SMEM((2,), jnp.float32)],
    )(pred, target)


def _sc_body(pred_hbm, target_hbm, loss_hbm, pos_hbm, px, pt, accv, sem):
    wid = lax.axis_index("s") * _NC + lax.axis_index("c")
    row0 = wid * _ROWS_PER_WORKER

    def chunk_body(j, carry):
        acc_loss, acc_pos = carry
        col0 = j * _CHUNK_COLS
        pltpu.sync_copy(
            pred_hbm.at[pl.ds(row0, _ROWS_PER_WORKER), pl.ds(col0, _CHUNK_COLS)],
            px)
        pltpu.sync_copy(
            target_hbm.at[pl.ds(row0, _ROWS_PER_WORKER), pl.ds(col0, _CHUNK_COLS)],
            pt)

        def vec_body(v, c2):
            al, ap = c2
            c = v * _LANES
            for r in range(_ROWS_PER_WORKER):
                x = px[r, pl.ds(c, _LANES)]
                lab = pt[r, pl.ds(c, _LANES)]
                s = x * (1.0 - 2.0 * lab)
                e = jnp.exp(-jnp.abs(x))
                inv = 1.0 / (1.0 + e)
                q = jnp.where(s >= 0.0, inv, 1.0 - inv)
                t = e / (2.0 + e)
                t2 = t * t
                poly = 1.0 + t2 * (1.0 / 3.0 + t2 * (1.0 / 5.0 + t2 * (
                    1.0 / 7.0 + t2 * (1.0 / 9.0))))
                sp = jnp.maximum(s, 0.0) + 2.0 * t * poly
                w = 0.75 - 0.5 * lab
                al = al + w * (q * q) * sp
                ap = ap + lab
            return (al, ap)

        return lax.fori_loop(0, _VEC_PER_CHUNK, vec_body, (acc_loss, acc_pos))

    zero = jnp.zeros((_LANES,), jnp.float32)
    acc_loss, acc_pos = lax.fori_loop(0, _N_CHUNKS, chunk_body, (zero, zero))
    accv[pl.ds(0, _LANES)] = acc_loss
    accv[pl.ds(_LANES, _LANES)] = acc_pos
    pltpu.sync_copy(accv.at[pl.ds(0, _LANES)],
                    loss_hbm.at[pl.ds(wid * _LANES, _LANES)])
    pltpu.sync_copy(accv.at[pl.ds(_LANES, _LANES)],
                    pos_hbm.at[pl.ds(wid * _LANES, _LANES)])


def _sc_partials(pred, target):
    mesh = plsc.VectorSubcoreMesh(
        core_axis_name="c", subcore_axis_name="s",
        num_cores=_NC, num_subcores=_NS)
    k = pl.kernel(
        _sc_body,
        out_type=(
            jax.ShapeDtypeStruct((_NW * _LANES,), jnp.float32),
            jax.ShapeDtypeStruct((_NW * _LANES,), jnp.float32),
        ),
        mesh=mesh,
        scratch_types=[
            pltpu.VMEM((_ROWS_PER_WORKER, _CHUNK_COLS), jnp.float32),
            pltpu.VMEM((_ROWS_PER_WORKER, _CHUNK_COLS), jnp.float32),
            pltpu.VMEM((2 * _LANES,), jnp.float32),
            pltpu.SemaphoreType.DMA,
        ],
    )
    return k(pred, target)


def kernel(pred, target):
    sc_loss, sc_pos = _sc_partials(pred, target)
    tc_out = _tc_partials(pred, target)
    loss_sum = tc_out[0] + jnp.sum(sc_loss)
    pos_num = jnp.maximum(tc_out[1] + jnp.sum(sc_pos), 1.0)
    return loss_sum / pos_num


# speedup vs baseline: 2.6657x; 1.2337x over previous
import functools

import jax
import jax.numpy as jnp
from jax import lax
from jax.experimental import pallas as pl
from jax.experimental.pallas import tpu as pltpu
from jax.experimental.pallas import tpu_sc as plsc

_ALPHA = 0.25

_ROWS = 1024
_COLS = 16384
_BLOCK_ROWS = 64

_SC_ROWS = 256

_NC, _NS, _LANES = 2, 16, 16
_NW = _NC * _NS
_ROWS_PER_WORKER = _SC_ROWS // _NW
_CHUNK_COLS = 2048
_N_CHUNKS = _COLS // _CHUNK_COLS
_VEC_PER_CHUNK = _CHUNK_COLS // _LANES

_LP = (1.4720650112765021e-06, 0.9998476974962275, -0.4973732161579119,
       0.31574731675788037, -0.19035433673294283, 0.08269123711132124,
       -0.017414077524226742)


def _tc_body(pred_ref, target_ref, out_ref, acc_ref):
    i = pl.program_id(0)

    @pl.when(i == 0)
    def _init():
        acc_ref[0] = 0.0
        acc_ref[1] = 0.0

    x = pred_ref[...]
    label = target_ref[...]
    s = x * (1.0 - 2.0 * label)
    w = (1.0 - _ALPHA) - (1.0 - 2.0 * _ALPHA) * label
    e = jnp.exp(-jnp.abs(s))
    denom = 1.0 + e
    inv = 1.0 / denom
    q = jnp.where(s >= 0.0, inv, 1.0 - inv)
    sp = jnp.maximum(s, 0.0) + jnp.log(denom)
    acc_ref[0] += jnp.sum(w * (q * q) * sp)
    acc_ref[1] += jnp.sum(label)

    @pl.when(i == pl.num_programs(0) - 1)
    def _finish():
        out_ref[0] = acc_ref[0]
        out_ref[1] = acc_ref[1]


def _tc_partials(pred, target):
    n_rows = _ROWS - _SC_ROWS
    grid = n_rows // _BLOCK_ROWS
    row0 = _SC_ROWS // _BLOCK_ROWS
    return pl.pallas_call(
        _tc_body,
        grid=(grid,),
        in_specs=[
            pl.BlockSpec((_BLOCK_ROWS, _COLS), lambda i: (i + row0, 0)),
            pl.BlockSpec((_BLOCK_ROWS, _COLS), lambda i: (i + row0, 0)),
        ],
        out_specs=pl.BlockSpec(memory_space=pltpu.SMEM),
        out_shape=jax.ShapeDtypeStruct((2,), jnp.float32),
        scratch_shapes=[pltpu.SMEM((2,), jnp.float32)],
    )(pred, target)


def _sc_body(pred_hbm, target_hbm, loss_hbm, pos_hbm,
             px0, pt0, px1, pt1, accv, sem0, sem1):
    wid = lax.axis_index("s") * _NC + lax.axis_index("c")
    row0 = wid * _ROWS_PER_WORKER
    bufs = ((px0, pt0, sem0), (px1, pt1, sem1))

    def issue(j):
        pxb, ptb, sem = bufs[j % 2]
        col0 = j * _CHUNK_COLS
        src = (pl.ds(row0, _ROWS_PER_WORKER), pl.ds(col0, _CHUNK_COLS))
        return (pltpu.async_copy(pred_hbm.at[src[0], src[1]], pxb, sem),
                pltpu.async_copy(target_hbm.at[src[0], src[1]], ptb, sem))

    acc_loss = jnp.zeros((_LANES,), jnp.float32)
    acc_pos = jnp.zeros((_LANES,), jnp.float32)
    handles = issue(0)
    for j in range(_N_CHUNKS):
        nxt = issue(j + 1) if j + 1 < _N_CHUNKS else None
        handles[0].wait()
        handles[1].wait()
        pxb, ptb, _ = bufs[j % 2]

        def vec_body(v, c2, pxb=pxb, ptb=ptb):
            al, ap = c2
            c = v * _LANES
            for r in range(_ROWS_PER_WORKER):
                x = pxb[r, pl.ds(c, _LANES)]
                lab = ptb[r, pl.ds(c, _LANES)]
                bl = lab > 0.0
                b = (x >= 0.0) != bl
                ax = jnp.abs(x)
                e = jnp.exp(-ax)
                inv = 1.0 / (1.0 + e)
                q = jnp.where(b, inv, 1.0 - inv)
                lp = _LP[0] + e * (_LP[1] + e * (_LP[2] + e * (_LP[3] + e * (
                    _LP[4] + e * (_LP[5] + e * _LP[6])))))
                sp = jnp.where(b, ax, 0.0) + lp
                w = jnp.where(bl, 0.25, 0.75)
                al = al + (w * sp) * (q * q)
                ap = ap + lab
            return (al, ap)

        acc_loss, acc_pos = lax.fori_loop(
            0, _VEC_PER_CHUNK, vec_body, (acc_loss, acc_pos))
        handles = nxt

    accv[pl.ds(0, _LANES)] = acc_loss
    accv[pl.ds(_LANES, _LANES)] = acc_pos
    pltpu.sync_copy(accv.at[pl.ds(0, _LANES)],
                    loss_hbm.at[pl.ds(wid * _LANES, _LANES)])
    pltpu.sync_copy(accv.at[pl.ds(_LANES, _LANES)],
                    pos_hbm.at[pl.ds(wid * _LANES, _LANES)])


def _sc_partials(pred, target):
    mesh = plsc.VectorSubcoreMesh(
        core_axis_name="c", subcore_axis_name="s",
        num_cores=_NC, num_subcores=_NS)
    k = pl.kernel(
        _sc_body,
        out_type=(
            jax.ShapeDtypeStruct((_NW * _LANES,), jnp.float32),
            jax.ShapeDtypeStruct((_NW * _LANES,), jnp.float32),
        ),
        mesh=mesh,
        scratch_types=[
            pltpu.VMEM((_ROWS_PER_WORKER, _CHUNK_COLS), jnp.float32),
            pltpu.VMEM((_ROWS_PER_WORKER, _CHUNK_COLS), jnp.float32),
            pltpu.VMEM((_ROWS_PER_WORKER, _CHUNK_COLS), jnp.float32),
            pltpu.VMEM((_ROWS_PER_WORKER, _CHUNK_COLS), jnp.float32),
            pltpu.VMEM((2 * _LANES,), jnp.float32),
            pltpu.SemaphoreType.DMA,
            pltpu.SemaphoreType.DMA,
        ],
    )
    return k(pred, target)


def kernel(pred, target):
    sc_loss, sc_pos = _sc_partials(pred, target)
    tc_out = _tc_partials(pred, target)
    loss_sum = tc_out[0] + jnp.sum(sc_loss)
    pos_num = jnp.maximum(tc_out[1] + jnp.sum(sc_pos), 1.0)
    return loss_sum / pos_num


# speedup vs baseline: 2.8122x; 1.0550x over previous
import functools

import jax
import jax.numpy as jnp
from jax import lax
from jax.experimental import pallas as pl
from jax.experimental.pallas import tpu as pltpu
from jax.experimental.pallas import tpu_sc as plsc

_ALPHA = 0.25

_ROWS = 1024
_COLS = 16384
_BLOCK_ROWS = 64

_SC_ROWS = 256

_NC, _NS, _LANES = 2, 16, 16
_NW = _NC * _NS
_ROWS_PER_WORKER = _SC_ROWS // _NW
_CHUNK_COLS = 2048
_N_CHUNKS = _COLS // _CHUNK_COLS
_VEC_PER_CHUNK = _CHUNK_COLS // _LANES

_LP = (1.4720650112765021e-06, 0.9998476974962275, -0.4973732161579119,
       0.31574731675788037, -0.19035433673294283, 0.08269123711132124,
       -0.017414077524226742)


def _tc_body(pred_ref, target_ref, out_ref, acc_ref):
    i = pl.program_id(0)

    @pl.when(i == 0)
    def _init():
        acc_ref[0] = 0.0
        acc_ref[1] = 0.0

    x = pred_ref[...]
    label = target_ref[...]
    s = x * (1.0 - 2.0 * label)
    w = (1.0 - _ALPHA) - (1.0 - 2.0 * _ALPHA) * label
    e = jnp.exp(-jnp.abs(s))
    denom = 1.0 + e
    inv = 1.0 / denom
    q = jnp.where(s >= 0.0, inv, 1.0 - inv)
    sp = jnp.maximum(s, 0.0) + jnp.log(denom)
    acc_ref[0] += jnp.sum(w * (q * q) * sp)
    acc_ref[1] += jnp.sum(label)

    @pl.when(i == pl.num_programs(0) - 1)
    def _finish():
        out_ref[0] = acc_ref[0]
        out_ref[1] = acc_ref[1]


def _tc_partials(pred, target):
    n_rows = _ROWS - _SC_ROWS
    grid = n_rows // _BLOCK_ROWS
    row0 = _SC_ROWS // _BLOCK_ROWS
    return pl.pallas_call(
        _tc_body,
        grid=(grid,),
        in_specs=[
            pl.BlockSpec((_BLOCK_ROWS, _COLS), lambda i: (i + row0, 0)),
            pl.BlockSpec((_BLOCK_ROWS, _COLS), lambda i: (i + row0, 0)),
        ],
        out_specs=pl.BlockSpec(memory_space=pltpu.SMEM),
        out_shape=jax.ShapeDtypeStruct((2,), jnp.float32),
        scratch_shapes=[pltpu.SMEM((2,), jnp.float32)],
    )(pred, target)


def _sc_body(pred_hbm, target_hbm, loss_hbm, pos_hbm,
             px0, pt0, px1, pt1, accv, sem0, sem1):
    wid = lax.axis_index("s") * _NC + lax.axis_index("c")
    row0 = wid * _ROWS_PER_WORKER
    bufs = ((px0, pt0, sem0), (px1, pt1, sem1))

    def issue(j, pxb, ptb, sem):
        col0 = j * _CHUNK_COLS
        pltpu.async_copy(
            pred_hbm.at[pl.ds(row0, _ROWS_PER_WORKER), pl.ds(col0, _CHUNK_COLS)],
            pxb, sem)
        pltpu.async_copy(
            target_hbm.at[pl.ds(row0, _ROWS_PER_WORKER), pl.ds(col0, _CHUNK_COLS)],
            ptb, sem)

    def drain(pxb, ptb, sem):
        pltpu.make_async_copy(
            pred_hbm.at[pl.ds(row0, _ROWS_PER_WORKER), pl.ds(0, _CHUNK_COLS)],
            pxb, sem).wait()
        pltpu.make_async_copy(
            target_hbm.at[pl.ds(row0, _ROWS_PER_WORKER), pl.ds(0, _CHUNK_COLS)],
            ptb, sem).wait()

    def compute(pxb, ptb, al, ap):
        def vec_body(v, c2):
            al, ap = c2
            c = v * _LANES
            for r in range(_ROWS_PER_WORKER):
                x = pxb[r, pl.ds(c, _LANES)]
                lab = ptb[r, pl.ds(c, _LANES)]
                bl = lab > 0.0
                b = (x >= 0.0) != bl
                ax = jnp.abs(x)
                e = jnp.exp(-ax)
                inv = 1.0 / (1.0 + e)
                q = jnp.where(b, inv, 1.0 - inv)
                lp = _LP[0] + e * (_LP[1] + e * (_LP[2] + e * (_LP[3] + e * (
                    _LP[4] + e * (_LP[5] + e * _LP[6])))))
                sp = jnp.where(b, ax, 0.0) + lp
                w = jnp.where(bl, 0.25, 0.75)
                al = al + (w * sp) * (q * q)
                ap = ap + lab
            return (al, ap)

        return lax.fori_loop(0, _VEC_PER_CHUNK, vec_body, (al, ap))

    issue(0, px0, pt0, sem0)
    issue(1, px1, pt1, sem1)

    def ring_body(i, carry):
        al, ap = carry
        g = 2 * i
        drain(px0, pt0, sem0)
        al, ap = compute(px0, pt0, al, ap)

        @pl.when(g + 2 < _N_CHUNKS)
        def _():
            issue(g + 2, px0, pt0, sem0)

        drain(px1, pt1, sem1)
        al, ap = compute(px1, pt1, al, ap)

        @pl.when(g + 3 < _N_CHUNKS)
        def _():
            issue(g + 3, px1, pt1, sem1)

        return (al, ap)

    zero = jnp.zeros((_LANES,), jnp.float32)
    acc_loss, acc_pos = lax.fori_loop(
        0, _N_CHUNKS // 2, ring_body, (zero, zero))

    accv[pl.ds(0, _LANES)] = acc_loss
    accv[pl.ds(_LANES, _LANES)] = acc_pos
    pltpu.sync_copy(accv.at[pl.ds(0, _LANES)],
                    loss_hbm.at[pl.ds(wid * _LANES, _LANES)])
    pltpu.sync_copy(accv.at[pl.ds(_LANES, _LANES)],
                    pos_hbm.at[pl.ds(wid * _LANES, _LANES)])


def _sc_partials(pred, target):
    mesh = plsc.VectorSubcoreMesh(
        core_axis_name="c", subcore_axis_name="s",
        num_cores=_NC, num_subcores=_NS)
    k = pl.kernel(
        _sc_body,
        out_type=(
            jax.ShapeDtypeStruct((_NW * _LANES,), jnp.float32),
            jax.ShapeDtypeStruct((_NW * _LANES,), jnp.float32),
        ),
        mesh=mesh,
        scratch_types=[
            pltpu.VMEM((_ROWS_PER_WORKER, _CHUNK_COLS), jnp.float32),
            pltpu.VMEM((_ROWS_PER_WORKER, _CHUNK_COLS), jnp.float32),
            pltpu.VMEM((_ROWS_PER_WORKER, _CHUNK_COLS), jnp.float32),
            pltpu.VMEM((_ROWS_PER_WORKER, _CHUNK_COLS), jnp.float32),
            pltpu.VMEM((2 * _LANES,), jnp.float32),
            pltpu.SemaphoreType.DMA,
            pltpu.SemaphoreType.DMA,
        ],
    )
    return k(pred, target)


def kernel(pred, target):
    sc_loss, sc_pos = _sc_partials(pred, target)
    tc_out = _tc_partials(pred, target)
    loss_sum = tc_out[0] + jnp.sum(sc_loss)
    pos_num = jnp.maximum(tc_out[1] + jnp.sum(sc_pos), 1.0)
    return loss_sum / pos_num
